# Initial kernel scaffold; baseline (speedup 1.0000x reference)
import jax
import jax.numpy as jnp
from jax.experimental import pallas as pl


def kernel(X, X_w, lbl_idx, lbl_w, embed_table, W_fine, b_fine, clf_u, clf_bias):
    raise NotImplementedError("write your pallas kernel here")


# speedup vs baseline: 1.8535x; 1.8535x over previous
import functools

import jax
import jax.numpy as jnp
from jax import lax
from jax.experimental import pallas as pl
from jax.experimental.pallas import tpu as pltpu
from jax.experimental.pallas import tpu_sc as plsc

_VOCAB = 1000000
_D = 64
_B = 1024
_L = 100000
_LNNZ = 20

_NC, _NS, _LANES = 2, 16, 16
_NW = _NC * _NS

_L_PAD = 100352
_LBL_PER_W = _L_PAD // _NW
_LBL_CHUNK = 32
_LBL_CHUNKS = _LBL_PER_W // _LBL_CHUNK
_LBL_G = _LBL_CHUNK * _LNNZ // 128
_LBL_G_PER_W = _LBL_PER_W * _LNNZ // 128

_DNNZ_PAD = 256
_DOC_PER_W = _B // _NW
_DOC_CHUNK = 2
_DOC_CHUNKS = _DOC_PER_W // _DOC_CHUNK
_DOC_G = _DOC_CHUNK * _DNNZ_PAD // 128
_DOC_G_PER_W = _DOC_PER_W * _DNNZ_PAD // 128


def _bag_body(tbl, lidx, lw, didx, dw, dout, lout, idx_v, rows_v, w_v, out_v, sem):
    wid = lax.axis_index("s") * _NC + lax.axis_index("c")

    def accumulate(nnz, base_k, accs):
        for n in range(nnz):
            k = base_k + n
            wv = plsc.load_gather(w_v, [jnp.full((_LANES,), k, jnp.int32)])
            accs = tuple(accs[c] + wv * rows_v[k, pl.ds(c * 16, 16)]
                         for c in range(4))
        return accs

    zeros4 = tuple(jnp.zeros((_LANES,), jnp.float32) for _ in range(4))

    def doc_chunk(ch, _):
        dbase = wid * _DOC_PER_W + ch * _DOC_CHUNK
        nidx = _DOC_CHUNK * _DNNZ_PAD
        pltpu.sync_copy(didx.at[pl.ds(dbase * _DNNZ_PAD, nidx)],
                        idx_v.at[pl.ds(0, nidx)])
        cps = [pltpu.async_copy(tbl.at[idx_v.at[pl.ds(j * 128, 128)]],
                                rows_v.at[pl.ds(j * 128, 128)], sem)
               for j in range(_DOC_G)]
        pltpu.sync_copy(dw.at[pl.ds(dbase * _DNNZ_PAD, _DOC_CHUNK * _DNNZ_PAD)],
                        w_v.at[pl.ds(0, _DOC_CHUNK * _DNNZ_PAD)])
        for cp in cps:
            cp.wait()

        def one_doc(i, _):
            def grp(g, accs):
                return accumulate(16, i * _DNNZ_PAD + g * 16, accs)
            accs = lax.fori_loop(0, _DNNZ_PAD // 16, grp, zeros4)
            for c in range(4):
                out_v[i, pl.ds(c * 16, 16)] = accs[c]
            return 0

        lax.fori_loop(0, _DOC_CHUNK, one_doc, 0)
        pltpu.sync_copy(out_v.at[pl.ds(0, _DOC_CHUNK)],
                        dout.at[pl.ds(dbase, _DOC_CHUNK)])
        return 0

    lax.fori_loop(0, _DOC_CHUNKS, doc_chunk, 0)

    def lbl_chunk(ch, _):
        lbase = wid * _LBL_PER_W + ch * _LBL_CHUNK
        nidx = _LBL_CHUNK * _LNNZ
        pltpu.sync_copy(lidx.at[pl.ds(lbase * _LNNZ, nidx)],
                        idx_v.at[pl.ds(0, nidx)])
        cps = [pltpu.async_copy(tbl.at[idx_v.at[pl.ds(j * 128, 128)]],
                                rows_v.at[pl.ds(j * 128, 128)], sem)
               for j in range(_LBL_G)]
        pltpu.sync_copy(lw.at[pl.ds(lbase * _LNNZ, _LBL_CHUNK * _LNNZ)],
                        w_v.at[pl.ds(0, _LBL_CHUNK * _LNNZ)])
        for cp in cps:
            cp.wait()

        def one_lbl(i, _):
            accs = accumulate(_LNNZ, i * _LNNZ, zeros4)
            for c in range(4):
                out_v[i, pl.ds(c * 16, 16)] = accs[c]
            return 0

        lax.fori_loop(0, _LBL_CHUNK, one_lbl, 0)
        pltpu.sync_copy(out_v.at[pl.ds(0, _LBL_CHUNK)],
                        lout.at[pl.ds(lbase, _LBL_CHUNK)])
        return 0

    lax.fori_loop(0, _LBL_CHUNKS, lbl_chunk, 0)


@functools.partial(
    pl.kernel,
    out_type=(jax.ShapeDtypeStruct((_B, _D), jnp.float32),
              jax.ShapeDtypeStruct((_L_PAD, _D), jnp.float32)),
    mesh=plsc.VectorSubcoreMesh(core_axis_name="c", subcore_axis_name="s"),
    scratch_types=[
        pltpu.VMEM((_LBL_CHUNK * _LNNZ,), jnp.int32),
        pltpu.VMEM((_LBL_G * 128, _D), jnp.float32),
        pltpu.VMEM((max(_LBL_CHUNK * _LNNZ, _DOC_CHUNK * _DNNZ_PAD),), jnp.float32),
        pltpu.VMEM((_LBL_CHUNK, _D), jnp.float32),
        pltpu.SemaphoreType.DMA,
    ],
    compiler_params=pltpu.CompilerParams(
        needs_layout_passes=False, use_tc_tiling_on_sc=False),
)
def _bag(tbl, lidx, lw, didx, dw, dout, lout, idx_v, rows_v, w_v, out_v, sem):
    _bag_body(tbl, lidx, lw, didx, dw, dout, lout, idx_v, rows_v, w_v, out_v, sem)


_LT = 2048
_GRID_L = _L_PAD // _LT


def _logits_body(doc_ref, wf_ref, bf_ref, lf_ref, cu_ref, cb_ref, out_ref, enc_scr):
    i = pl.program_id(0)

    @pl.when(i == 0)
    def _():
        d = doc_ref[...]
        h = jnp.dot(d, wf_ref[...], preferred_element_type=jnp.float32,
                    precision=lax.Precision.HIGHEST) + bf_ref[...]
        enc_scr[...] = d + jnp.maximum(h, 0.0)

    w = lf_ref[...] + cu_ref[...]
    out_ref[...] = lax.dot_general(
        enc_scr[...], w, (((1,), (1,)), ((), ())),
        precision=lax.Precision.HIGHEST,
        preferred_element_type=jnp.float32) + cb_ref[...]


def _logits(doc_emb, W_fine, b_fine2, label_fts, cu_p, cb_p):
    return pl.pallas_call(
        _logits_body,
        grid=(_GRID_L,),
        in_specs=[
            pl.BlockSpec((_B, _D), lambda i: (0, 0)),
            pl.BlockSpec((_D, _D), lambda i: (0, 0)),
            pl.BlockSpec((1, _D), lambda i: (0, 0)),
            pl.BlockSpec((_LT, _D), lambda i: (i, 0)),
            pl.BlockSpec((_LT, _D), lambda i: (i, 0)),
            pl.BlockSpec((1, _LT), lambda i: (0, i)),
        ],
        out_specs=pl.BlockSpec((_B, _LT), lambda i: (0, i)),
        out_shape=jax.ShapeDtypeStruct((_B, _L), jnp.float32),
        scratch_shapes=[pltpu.VMEM((_B, _D), jnp.float32)],
    )(doc_emb, W_fine, b_fine2, label_fts, cu_p, cb_p)


def kernel(X, X_w, lbl_idx, lbl_w, embed_table, W_fine, b_fine, clf_u, clf_bias):
    Xp = jnp.pad(X, ((0, 0), (0, _DNNZ_PAD - X.shape[1])), constant_values=_VOCAB)
    Xwp = jnp.pad(X_w, ((0, 0), (0, _DNNZ_PAD - X_w.shape[1])))
    didx = Xp.reshape(-1)
    dw = Xwp.reshape(-1)
    lpad = _L_PAD - _L
    lidx = jnp.pad(lbl_idx, ((0, lpad), (0, 0)), constant_values=_VOCAB).reshape(-1)
    lw = jnp.pad(lbl_w, ((0, lpad), (0, 0))).reshape(-1)

    doc_emb, label_fts = _bag(embed_table, lidx, lw, didx, dw)

    cu_p = jnp.pad(clf_u, ((0, lpad), (0, 0)))
    cb_p = jnp.pad(clf_bias, (0, lpad)).reshape(1, -1)
    return _logits(doc_emb, W_fine, b_fine.reshape(1, -1), label_fts, cu_p, cb_p)
